# baseline (speedup 1.0000x reference)
import jax
import jax.numpy as jnp
from jax.experimental import pallas as pl
from jax.experimental.pallas import tpu as pltpu


def _interp_kernel(x1_ref, x2_ref, p2_ref, w_ref, b_ref, out_ref):
    x1 = x1_ref[0]
    x2 = x2_ref[0]
    S = x2.shape[1]
    NT = x1.shape[1]

    mm = jax.lax.dot_general(x2, x1, (((0,), (0,)), ((), ())),
                             preferred_element_type=jnp.float32)
    s1 = (x1[0] * x1[0] + x1[1] * x1[1]) + x1[2] * x1[2]
    s2 = (x2[0] * x2[0] + x2[1] * x2[1]) + x2[2] * x2[2]
    d = -2.0 * mm
    d = d + s1[None, :]
    d = d + s2[:, None]

    iota = jax.lax.broadcasted_iota(jnp.int32, (S, NT), 0)
    u = jnp.zeros((S, NT), jnp.float32)
    recips = []
    for k in range(3):
        m = jnp.min(d, axis=0, keepdims=True)
        i = jnp.min(jnp.where(d == m, iota, S), axis=0, keepdims=True)
        r = 1.0 / (m + 1e-8)
        recips.append(r)
        sel = iota == i
        u = jnp.where(sel, r, u)
        if k < 2:
            d = jnp.where(sel, jnp.float32(jnp.inf), d)

    norm = (recips[0] + recips[1]) + recips[2]
    a = u * (1.0 / norm)

    interp = jnp.dot(p2_ref[0], a, preferred_element_type=jnp.float32)
    out = jnp.dot(w_ref[...], interp, preferred_element_type=jnp.float32)
    out_ref[0] = out + b_ref[:, 0:1]


def kernel(xyz1, xyz2, points2, W, b):
    B, _, N = xyz1.shape
    S = xyz2.shape[2]
    Cout, Cin = W.shape
    NT = 1024
    grid = (B, N // NT)
    return pl.pallas_call(
        _interp_kernel,
        grid=grid,
        in_specs=[
            pl.BlockSpec((1, 3, NT), lambda bb, nn: (bb, 0, nn)),
            pl.BlockSpec((1, 3, S), lambda bb, nn: (bb, 0, 0)),
            pl.BlockSpec((1, Cin, S), lambda bb, nn: (bb, 0, 0)),
            pl.BlockSpec((Cout, Cin), lambda bb, nn: (0, 0)),
            pl.BlockSpec((Cout, 128), lambda bb, nn: (0, 0)),
        ],
        out_specs=pl.BlockSpec((1, Cout, NT), lambda bb, nn: (bb, 0, nn)),
        out_shape=jax.ShapeDtypeStruct((B, Cout, N), jnp.float32),
        compiler_params=pltpu.CompilerParams(
            dimension_semantics=("parallel", "parallel"),
        ),
    )(xyz1, xyz2, points2, W, jnp.broadcast_to(b[:, None], (Cout, 128)))


# speedup vs baseline: 1.0441x; 1.0441x over previous
import jax
import jax.numpy as jnp
from jax.experimental import pallas as pl
from jax.experimental.pallas import tpu as pltpu


def _interp_kernel(x1_ref, x2_ref, p2_ref, w_ref, b_ref, out_ref):
    x1 = x1_ref[0]
    x2 = x2_ref[0]
    S = x2.shape[1]
    NT = x1.shape[1]

    mm = jax.lax.dot_general(x2, x1, (((0,), (0,)), ((), ())),
                             preferred_element_type=jnp.float32)
    s1 = (x1[0] * x1[0] + x1[1] * x1[1]) + x1[2] * x1[2]
    s2 = (x2[0] * x2[0] + x2[1] * x2[1]) + x2[2] * x2[2]
    d = -2.0 * mm
    d = d + s1[None, :]
    d = d + s2[:, None]

    iota = jax.lax.broadcasted_iota(jnp.int32, (S, NT), 0)
    u = jnp.zeros((S, NT), jnp.float32)
    recips = []
    for k in range(3):
        m = jnp.min(d, axis=0, keepdims=True)
        i = jnp.min(jnp.where(d == m, iota, S), axis=0, keepdims=True)
        r = 1.0 / (m + 1e-8)
        recips.append(r)
        sel = iota == i
        u = jnp.where(sel, r, u)
        if k < 2:
            d = jnp.where(sel, jnp.float32(jnp.inf), d)

    norm = (recips[0] + recips[1]) + recips[2]
    a = u * (1.0 / norm)

    interp = jnp.dot(p2_ref[0], a, preferred_element_type=jnp.float32)
    out = jnp.dot(w_ref[...], interp, preferred_element_type=jnp.float32)
    out_ref[0] = out + b_ref[:, 0:1]


def kernel(xyz1, xyz2, points2, W, b):
    B, _, N = xyz1.shape
    S = xyz2.shape[2]
    Cout, Cin = W.shape
    NT = 2048
    grid = (B, N // NT)
    return pl.pallas_call(
        _interp_kernel,
        grid=grid,
        in_specs=[
            pl.BlockSpec((1, 3, NT), lambda bb, nn: (bb, 0, nn)),
            pl.BlockSpec((1, 3, S), lambda bb, nn: (bb, 0, 0)),
            pl.BlockSpec((1, Cin, S), lambda bb, nn: (bb, 0, 0)),
            pl.BlockSpec((Cout, Cin), lambda bb, nn: (0, 0)),
            pl.BlockSpec((Cout, 128), lambda bb, nn: (0, 0)),
        ],
        out_specs=pl.BlockSpec((1, Cout, NT), lambda bb, nn: (bb, 0, nn)),
        out_shape=jax.ShapeDtypeStruct((B, Cout, N), jnp.float32),
        compiler_params=pltpu.CompilerParams(
            dimension_semantics=("parallel", "parallel"),
        ),
    )(xyz1, xyz2, points2, W, jnp.broadcast_to(b[:, None], (Cout, 128)))
